# baseline (speedup 1.0000x reference)
import functools

import jax
import jax.numpy as jnp
from jax import lax
from jax.experimental import pallas as pl
from jax.experimental.pallas import tpu as pltpu
from jax.experimental.pallas import tpu_sc as plsc

B = 32
LQ = 32
N = 10000
LC = 32
K = 50
KP = 64
D = 128
V = 30000
VP = 30080
N2 = 10240
KLC = K * LC

NC = 2
NS = 16
NW = NC * NS

_f32 = jnp.float32
_i32 = jnp.int32


def _wid():
    return lax.axis_index("s") * NC + lax.axis_index("c")


def _sc_mesh():
    return plsc.VectorSubcoreMesh(
        core_axis_name="c", subcore_axis_name="s",
        num_cores=NC, num_subcores=NS)


def _g1_body(q0f_hbm, q1f_hbm, eir_hbm, erc_hbm, erc2_hbm,
             qeir_hbm, qerca_hbm, qercb_hbm,
             t0v, t1v, ebuf):
    b = _wid()
    pltpu.sync_copy(q0f_hbm.at[pl.ds(b * LQ, LQ)], t0v)
    pltpu.sync_copy(q1f_hbm.at[pl.ds(b * LQ, LQ)], t1v)
    pltpu.sync_copy(eir_hbm.at[t0v], ebuf)
    pltpu.sync_copy(ebuf, qeir_hbm.at[b])
    pltpu.sync_copy(erc_hbm.at[t0v], ebuf)
    pltpu.sync_copy(ebuf, qerca_hbm.at[b])
    pltpu.sync_copy(erc2_hbm.at[t1v], ebuf)
    pltpu.sync_copy(ebuf, qercb_hbm.at[b])


def _g1(q0f, q1f, emb_ir, emb_rc, emb_rc2):
    out_type = (jax.ShapeDtypeStruct((B, LQ, D), _f32),
                jax.ShapeDtypeStruct((B, LQ, D), _f32),
                jax.ShapeDtypeStruct((B, LQ, D), _f32))
    return pl.kernel(
        _g1_body, out_type=out_type, mesh=_sc_mesh(),
        compiler_params=pltpu.CompilerParams(use_tc_tiling_on_sc=False),
        scratch_types=[pltpu.VMEM((LQ,), _i32),
                       pltpu.VMEM((LQ,), _i32),
                       pltpu.VMEM((LQ, D), _f32)],
    )(q0f, q1f, emb_ir, emb_rc, emb_rc2)


def _t0_body(qeir, qerca, qercb, qlen, wq, ws, qv_out, vs_out):
    R = B * LQ
    bi_b = lax.broadcasted_iota(_i32, (B, R), 0)
    bi_r = lax.broadcasted_iota(_i32, (B, R), 1) // LQ
    P = (bi_b == bi_r).astype(_f32)
    qlf = qlen[...].astype(_f32)
    ql_row = lax.dot_general(P, qlf, (((0,), (1,)), ((), ())), precision=lax.Precision.HIGHEST)
    tmod = (lax.broadcasted_iota(_i32, (R, 1), 0) % LQ).astype(_f32)
    m = (tmod < ql_row).astype(_f32)
    ql_col = lax.dot_general(P, m, (((1,), (0,)), ((), ())), precision=lax.Precision.HIGHEST)
    den = jnp.maximum(ql_col, 1.0)
    qv = lax.dot_general(P, qeir[...] * m, (((1,), (0,)), ((), ())), precision=lax.Precision.HIGHEST) / den
    qv_out[...] = qv
    erc = qerca[...] + qercb[...]
    qm2 = lax.dot_general(P, erc * m, (((1,), (0,)), ((), ())), precision=lax.Precision.HIGHEST) / den
    qh = jnp.dot(qm2, wq[...], preferred_element_type=_f32, precision=lax.Precision.HIGHEST)
    vs_out[...] = qh * ws[...]


def _t0(qeir, qerca, qercb, qlen, Wq, w_s):
    return pl.pallas_call(
        _t0_body,
        out_shape=(jax.ShapeDtypeStruct((B, D), _f32),
                   jax.ShapeDtypeStruct((B, D), _f32)),
    )(qeir.reshape(B * LQ, D), qerca.reshape(B * LQ, D),
      qercb.reshape(B * LQ, D), qlen.reshape(1, B),
      Wq, w_s.reshape(1, D))


_VB = 640


def _t1_body(emb, qv, out):
    i = pl.program_id(0)
    s = lax.dot_general(emb[...], qv[...], (((1,), (1,)), ((), ())), precision=lax.Precision.HIGHEST)
    row = lax.broadcasted_iota(_i32, (_VB, 1), 0) + i * _VB
    out[...] = jnp.where(row < V, s, 0.0)


def _t1(emb_ir, qv):
    return pl.pallas_call(
        _t1_body,
        grid=(VP // _VB,),
        in_specs=[pl.BlockSpec((_VB, D), lambda i: (i, 0)),
                  pl.BlockSpec((B, D), lambda i: (0, 0))],
        out_specs=pl.BlockSpec((_VB, B), lambda i: (i, 0)),
        out_shape=jax.ShapeDtypeStruct((VP, B), _f32),
    )(emb_ir, qv)


_NSW = N2 // NW
_NSUB = _NSW // 4


def _g2_body(c0f_hbm, clen_hbm, sT_hbm, out_hbm,
             cbuf, clv, tokb, srows, sbuf, sem0, sem1):
    w = _wid()
    sems = (sem0, sem1)

    pltpu.sync_copy(c0f_hbm.at[pl.ds(w * _NSW * LC, _NSW * LC)], cbuf)
    pltpu.sync_copy(clen_hbm.at[pl.ds(w * _NSW, _NSW)], clv.at[pl.ds(0, _NSW)])

    def build_and_issue(s, par):
        clvvec = clv[pl.ds(s * 4, 16)]
        for r in range(4):
            cls = jnp.broadcast_to(clvvec[r], (16,))
            for k in range(2):
                tl = lax.iota(_i32, 16) + 16 * k
                tok = cbuf[pl.ds((s * 4 + r) * LC + 16 * k, 16)]
                tokb[par, pl.ds(r * LC + 16 * k, 16)] = (
                    jnp.where(tl < cls, tok, V))
        pltpu.async_copy(sT_hbm.at[tokb.at[par]], srows.at[par], sems[par])

    def wait(par):
        pltpu.make_async_copy(
            sT_hbm.at[tokb.at[par]], srows.at[par], sems[par]).wait()

    def accumulate(s, par):
        clvvec = clv[pl.ds(s * 4, 16)]
        for r in range(4):
            r0 = r * LC
            a0 = srows[par, r0, pl.ds(0, 16)]
            a1 = srows[par, r0, pl.ds(16, 16)]
            for t in range(1, LC):
                a0 = a0 + srows[par, r0 + t, pl.ds(0, 16)]
                a1 = a1 + srows[par, r0 + t, pl.ds(16, 16)]
            clf = jnp.broadcast_to(clvvec[r], (16,)).astype(_f32)
            sbuf[s * 4 + r, pl.ds(0, 16)] = a0 / clf
            sbuf[s * 4 + r, pl.ds(16, 16)] = a1 / clf

    build_and_issue(0, 0)

    @pl.loop(0, _NSUB // 2)
    def _i(i):
        for par in range(2):
            s = i * 2 + par

            @pl.when(s < _NSUB - 1)
            def _():
                build_and_issue(s + 1, 1 - par)

            wait(par)
            accumulate(s, par)

    pltpu.sync_copy(sbuf, out_hbm.at[pl.ds(w * _NSW, _NSW)])


def _g2(c0f, clen, sT):
    return pl.kernel(
        _g2_body, out_type=jax.ShapeDtypeStruct((N2, B), _f32),
        mesh=_sc_mesh(),
        compiler_params=pltpu.CompilerParams(use_tc_tiling_on_sc=False),
        scratch_types=[pltpu.VMEM((_NSW * LC,), _i32),
                       pltpu.VMEM((_NSW + 16,), _i32),
                       pltpu.VMEM((2, 128), _i32),
                       pltpu.VMEM((2, 128, B), _f32),
                       pltpu.VMEM((_NSW, B), _f32),
                       pltpu.SemaphoreType.DMA,
                       pltpu.SemaphoreType.DMA],
    )(c0f, clen, sT)


def _t2_body(sT, out, s_scr, idx_scr):
    k = pl.program_id(0)
    bi0 = lax.broadcasted_iota(_i32, (N2, B), 0)

    @pl.when(k == 0)
    def _():
        s_scr[...] = jnp.where(bi0 < N, sT[...], -1e30)

    s = s_scr[...]
    m = jnp.max(s, axis=0, keepdims=True)
    idx = jnp.min(jnp.where(s == m, bi0, N2), axis=0, keepdims=True)
    idx_scr[pl.ds(k, 1), :] = idx
    s_scr[...] = jnp.where(bi0 == idx, -jnp.inf, s)

    @pl.when(k == KP - 1)
    def _():
        idf = idx_scr[...].astype(_f32)
        e0 = lax.broadcasted_iota(_i32, (KP, KP), 0)
        e1 = lax.broadcasted_iota(_i32, (KP, KP), 1)
        eye = (e0 == e1).astype(_f32)
        tr = lax.dot_general(idf, eye, (((0,), (0,)), ((), ())), precision=lax.Precision.HIGHEST)
        out[...] = tr.astype(_i32)


def _t2(sT):
    return pl.pallas_call(
        _t2_body,
        grid=(KP,),
        in_specs=[pl.BlockSpec((N2, B), lambda k: (0, 0))],
        out_specs=pl.BlockSpec((B, KP), lambda k: (0, 0)),
        out_shape=jax.ShapeDtypeStruct((B, KP), _i32),
        scratch_shapes=[pltpu.VMEM((N2, B), _f32),
                        pltpu.VMEM((KP, B), _i32)],
    )(sT)


def _g3_body(topk_hbm, clenr_hbm, c0_hbm, c1_hbm, erc_hbm, erc2_hbm,
             ea_hbm, eb_hbm, lensr_hbm,
             sidv, cb0, cb1, clrb, ebuf):
    b = _wid()
    pltpu.sync_copy(topk_hbm.at[b], sidv)
    pltpu.sync_copy(c0_hbm.at[sidv], cb0)
    pltpu.sync_copy(c1_hbm.at[sidv], cb1)
    pltpu.sync_copy(clenr_hbm.at[sidv], clrb)
    pltpu.sync_copy(clrb, lensr_hbm.at[b])

    @pl.loop(0, K)
    def _j(j):
        pltpu.sync_copy(erc_hbm.at[cb0.at[j]], ebuf)
        pltpu.sync_copy(ebuf, ea_hbm.at[b, pl.ds(j * LC, LC)])
        pltpu.sync_copy(erc2_hbm.at[cb1.at[j]], ebuf)
        pltpu.sync_copy(ebuf, eb_hbm.at[b, pl.ds(j * LC, LC)])


def _g3(topk, clen_rep, c0, c1, emb_rc, emb_rc2):
    out_type = (jax.ShapeDtypeStruct((B, KLC, D), _f32),
                jax.ShapeDtypeStruct((B, KLC, D), _f32),
                jax.ShapeDtypeStruct((B, KP, 16), _i32))
    return pl.kernel(
        _g3_body, out_type=out_type, mesh=_sc_mesh(),
        compiler_params=pltpu.CompilerParams(use_tc_tiling_on_sc=False),
        scratch_types=[pltpu.VMEM((KP,), _i32),
                       pltpu.VMEM((KP, LC), _i32),
                       pltpu.VMEM((KP, LC), _i32),
                       pltpu.VMEM((KP, 16), _i32),
                       pltpu.VMEM((LC, D), _f32)],
    )(topk, clen_rep, c0, c1, emb_rc, emb_rc2)


def _t3_body(ea, eb, w, vs, lens, a, out):
    h = ea[0] + eb[0]
    hw = jax.nn.relu(jnp.dot(h, w[...], preferred_element_type=_f32, precision=lax.Precision.HIGHEST))
    lg = jnp.sum(hw * vs[0], axis=1, keepdims=True)

    c1 = lax.broadcasted_iota(_i32, (1, KP), 1)
    lens50 = jnp.where(c1 < K, lens[0].astype(_f32), 0.0)
    u0 = lax.broadcasted_iota(_i32, (KP, KP), 0)
    u1 = lax.broadcasted_iota(_i32, (KP, KP), 1)
    ut = (u0 < u1).astype(_f32)
    off = lax.dot_general(lens50, ut, (((1,), (0,)), ((), ())), precision=lax.Precision.HIGHEST)

    r0 = lax.broadcasted_iota(_i32, (KLC, KP), 0) // LC
    rc = lax.broadcasted_iota(_i32, (KLC, KP), 1)
    P2 = (r0 == rc).astype(_f32)
    lens_row = lax.dot_general(P2, lens50, (((1,), (1,)), ((), ())), precision=lax.Precision.HIGHEST)
    off_row = lax.dot_general(P2, off, (((1,), (1,)), ((), ())), precision=lax.Precision.HIGHEST)
    tmod = (lax.broadcasted_iota(_i32, (KLC, 1), 0) % LC).astype(_f32)
    valid = tmod < lens_row
    pos = off_row + tmod

    ts = a[0, 0, 0] % KLC
    tsf = ts.astype(_f32)
    ctx = jnp.sum(lens50)
    hit = jnp.logical_and(valid, pos == tsf)
    pick = jnp.where(tsf < ctx, jnp.sum(jnp.where(hit, lg, 0.0)), -1e9)
    lm = jnp.max(jnp.where(valid, lg, -1e30))
    lse = lm + jnp.log(jnp.sum(jnp.where(valid, jnp.exp(lg - lm), 0.0)))

    bidx = pl.program_id(0)

    @pl.when(bidx == 0)
    def _():
        out[0, 0] = 0.0

    out[0, 0] += (lse - pick) * (1.0 / B)


def _t3(ea, eb, W, vs, lens, a):
    return pl.pallas_call(
        _t3_body,
        grid=(B,),
        in_specs=[pl.BlockSpec((1, KLC, D), lambda b: (b, 0, 0)),
                  pl.BlockSpec((1, KLC, D), lambda b: (b, 0, 0)),
                  pl.BlockSpec((D, D), lambda b: (0, 0)),
                  pl.BlockSpec((1, 1, D), lambda b: (b, 0, 0)),
                  pl.BlockSpec((1, 1, KP), lambda b: (b, 0, 0)),
                  pl.BlockSpec((1, 1, 16), lambda b: (b, 0, 0))],
        out_specs=pl.BlockSpec(memory_space=pltpu.SMEM),
        out_shape=jax.ShapeDtypeStruct((1, 1), _f32),
    )(ea, eb, W, vs.reshape(B, 1, D), lens.reshape(B, 1, KP),
      a.reshape(B, 1, 16))


def kernel(q, c, a, qlen, clen, alen, emb_ir, emb_rc, emb_rc2,
           W, Wq, w_s, w_e):
    q0f = q[:, :, 0].reshape(-1)
    q1f = q[:, :, 1].reshape(-1)
    c0 = c[:, :, 0]
    c1 = c[:, :, 1]
    qeir, qerca, qercb = _g1(q0f, q1f, emb_ir, emb_rc, emb_rc2)
    qv, vs = _t0(qeir, qerca, qercb, qlen, Wq, w_s)
    sT = _t1(emb_ir, qv)
    c0p = jnp.pad(c0, ((0, N2 - N), (0, 0)))
    clenp = jnp.pad(clen, (0, N2 - N), constant_values=1)
    scoresT = _g2(c0p.reshape(-1), clenp, sT)
    topk = _t2(scoresT)
    clen_rep = jnp.broadcast_to(clen[:, None], (N, 16))
    ea, eb, lensr = _g3(topk, clen_rep, c0, c1, emb_rc, emb_rc2)
    loss = _t3(ea, eb, W, vs, lensr[:, :, 0], a)
    return loss.reshape(())


# speedup vs baseline: 3.7165x; 3.7165x over previous
import functools

import jax
import jax.numpy as jnp
from jax import lax
from jax.experimental import pallas as pl
from jax.experimental.pallas import tpu as pltpu
from jax.experimental.pallas import tpu_sc as plsc

B = 32
LQ = 32
N = 10000
LC = 32
K = 50
KP = 64
D = 128
V = 30000
VP = 30080
N2 = 10240
KLC = K * LC

NC = 2
NS = 16
NW = NC * NS

_f32 = jnp.float32
_i32 = jnp.int32


def _wid():
    return lax.axis_index("s") * NC + lax.axis_index("c")


def _sc_mesh():
    return plsc.VectorSubcoreMesh(
        core_axis_name="c", subcore_axis_name="s",
        num_cores=NC, num_subcores=NS)


def _g1_body(q0f_hbm, q1f_hbm, eir_hbm, erc_hbm, erc2_hbm,
             qeir_hbm, qerca_hbm, qercb_hbm,
             t0v, t1v, ebuf):
    b = _wid()
    pltpu.sync_copy(q0f_hbm.at[pl.ds(b * LQ, LQ)], t0v)
    pltpu.sync_copy(q1f_hbm.at[pl.ds(b * LQ, LQ)], t1v)
    pltpu.sync_copy(eir_hbm.at[t0v], ebuf)
    pltpu.sync_copy(ebuf, qeir_hbm.at[b])
    pltpu.sync_copy(erc_hbm.at[t0v], ebuf)
    pltpu.sync_copy(ebuf, qerca_hbm.at[b])
    pltpu.sync_copy(erc2_hbm.at[t1v], ebuf)
    pltpu.sync_copy(ebuf, qercb_hbm.at[b])


def _g1(q0f, q1f, emb_ir, emb_rc, emb_rc2):
    out_type = (jax.ShapeDtypeStruct((B, LQ, D), _f32),
                jax.ShapeDtypeStruct((B, LQ, D), _f32),
                jax.ShapeDtypeStruct((B, LQ, D), _f32))
    return pl.kernel(
        _g1_body, out_type=out_type, mesh=_sc_mesh(),
        compiler_params=pltpu.CompilerParams(use_tc_tiling_on_sc=False),
        scratch_types=[pltpu.VMEM((LQ,), _i32),
                       pltpu.VMEM((LQ,), _i32),
                       pltpu.VMEM((LQ, D), _f32)],
    )(q0f, q1f, emb_ir, emb_rc, emb_rc2)


def _t0_body(qeir, qerca, qercb, qlen, wq, ws, qv_out, vs_out):
    R = B * LQ
    bi_b = lax.broadcasted_iota(_i32, (B, R), 0)
    bi_r = lax.broadcasted_iota(_i32, (B, R), 1) // LQ
    P = (bi_b == bi_r).astype(_f32)
    qlf = qlen[...].astype(_f32)
    ql_row = lax.dot_general(P, qlf, (((0,), (1,)), ((), ())), precision=lax.Precision.HIGHEST)
    tmod = (lax.broadcasted_iota(_i32, (R, 1), 0) % LQ).astype(_f32)
    m = (tmod < ql_row).astype(_f32)
    ql_col = lax.dot_general(P, m, (((1,), (0,)), ((), ())), precision=lax.Precision.HIGHEST)
    den = jnp.maximum(ql_col, 1.0)
    qv = lax.dot_general(P, qeir[...] * m, (((1,), (0,)), ((), ())), precision=lax.Precision.HIGHEST) / den
    qv_out[...] = qv
    erc = qerca[...] + qercb[...]
    qm2 = lax.dot_general(P, erc * m, (((1,), (0,)), ((), ())), precision=lax.Precision.HIGHEST) / den
    qh = jnp.dot(qm2, wq[...], preferred_element_type=_f32, precision=lax.Precision.HIGHEST)
    vs_out[...] = qh * ws[...]


def _t0(qeir, qerca, qercb, qlen, Wq, w_s):
    return pl.pallas_call(
        _t0_body,
        out_shape=(jax.ShapeDtypeStruct((B, D), _f32),
                   jax.ShapeDtypeStruct((B, D), _f32)),
    )(qeir.reshape(B * LQ, D), qerca.reshape(B * LQ, D),
      qercb.reshape(B * LQ, D), qlen.reshape(1, B),
      Wq, w_s.reshape(1, D))


_VB = 640


def _t1_body(emb, qv, out):
    i = pl.program_id(0)
    s = lax.dot_general(emb[...], qv[...], (((1,), (1,)), ((), ())), precision=lax.Precision.HIGHEST)
    row = lax.broadcasted_iota(_i32, (_VB, 1), 0) + i * _VB
    out[...] = jnp.where(row < V, s, 0.0)


def _t1(emb_ir, qv):
    return pl.pallas_call(
        _t1_body,
        grid=(VP // _VB,),
        in_specs=[pl.BlockSpec((_VB, D), lambda i: (i, 0)),
                  pl.BlockSpec((B, D), lambda i: (0, 0))],
        out_specs=pl.BlockSpec((_VB, B), lambda i: (i, 0)),
        out_shape=jax.ShapeDtypeStruct((VP, B), _f32),
    )(emb_ir, qv)


_NSW = N2 // NW
_NSUB = _NSW // 4


def _g2_body(c0f_hbm, clen_hbm, sT_hbm, out_hbm,
             cbuf, clv, tokb, srows, sbuf, sT_sh, sem0, sem1):
    w = _wid()
    sems = (sem0, sem1)

    @pl.when(lax.axis_index("s") == 0)
    def _():
        pltpu.sync_copy(sT_hbm, sT_sh)
    plsc.subcore_barrier()

    pltpu.sync_copy(c0f_hbm.at[pl.ds(w * _NSW * LC, _NSW * LC)], cbuf)
    pltpu.sync_copy(clen_hbm.at[pl.ds(w * _NSW, _NSW)], clv.at[pl.ds(0, _NSW)])

    def build_and_issue(s, par):
        clvvec = clv[pl.ds(s * 4, 16)]
        for r in range(4):
            cls = jnp.broadcast_to(clvvec[r], (16,))
            for k in range(2):
                tl = lax.iota(_i32, 16) + 16 * k
                tok = cbuf[pl.ds((s * 4 + r) * LC + 16 * k, 16)]
                tokb[par, pl.ds(r * LC + 16 * k, 16)] = (
                    jnp.where(tl < cls, tok, V))
        pltpu.async_copy(sT_sh.at[tokb.at[par]], srows.at[par], sems[par])

    def wait(par):
        pltpu.make_async_copy(
            sT_sh.at[tokb.at[par]], srows.at[par], sems[par]).wait()

    def accumulate(s, par):
        clvvec = clv[pl.ds(s * 4, 16)]
        for r in range(4):
            r0 = r * LC
            a0 = srows[par, r0, pl.ds(0, 16)]
            a1 = srows[par, r0, pl.ds(16, 16)]
            for t in range(1, LC):
                a0 = a0 + srows[par, r0 + t, pl.ds(0, 16)]
                a1 = a1 + srows[par, r0 + t, pl.ds(16, 16)]
            clf = jnp.broadcast_to(clvvec[r], (16,)).astype(_f32)
            sbuf[s * 4 + r, pl.ds(0, 16)] = a0 / clf
            sbuf[s * 4 + r, pl.ds(16, 16)] = a1 / clf

    build_and_issue(0, 0)

    @pl.loop(0, _NSUB // 2)
    def _i(i):
        for par in range(2):
            s = i * 2 + par

            @pl.when(s < _NSUB - 1)
            def _():
                build_and_issue(s + 1, 1 - par)

            wait(par)
            accumulate(s, par)

    pltpu.sync_copy(sbuf, out_hbm.at[pl.ds(w * _NSW, _NSW)])


def _g2(c0f, clen, sT):
    return pl.kernel(
        _g2_body, out_type=jax.ShapeDtypeStruct((N2, B), _f32),
        mesh=_sc_mesh(),
        compiler_params=pltpu.CompilerParams(use_tc_tiling_on_sc=False),
        scratch_types=[pltpu.VMEM((_NSW * LC,), _i32),
                       pltpu.VMEM((_NSW + 16,), _i32),
                       pltpu.VMEM((2, 128), _i32),
                       pltpu.VMEM((2, 128, B), _f32),
                       pltpu.VMEM((_NSW, B), _f32),
                       pltpu.VMEM_SHARED((VP, B), _f32),
                       pltpu.SemaphoreType.DMA,
                       pltpu.SemaphoreType.DMA],
    )(c0f, clen, sT)


def _t2_body(sT, out, s_scr, idx_scr):
    k = pl.program_id(0)
    bi0 = lax.broadcasted_iota(_i32, (N2, B), 0)

    @pl.when(k == 0)
    def _():
        s_scr[...] = jnp.where(bi0 < N, sT[...], -1e30)

    s = s_scr[...]
    m = jnp.max(s, axis=0, keepdims=True)
    idx = jnp.min(jnp.where(s == m, bi0, N2), axis=0, keepdims=True)
    idx_scr[pl.ds(k, 1), :] = idx
    s_scr[...] = jnp.where(bi0 == idx, -jnp.inf, s)

    @pl.when(k == KP - 1)
    def _():
        idf = idx_scr[...].astype(_f32)
        e0 = lax.broadcasted_iota(_i32, (KP, KP), 0)
        e1 = lax.broadcasted_iota(_i32, (KP, KP), 1)
        eye = (e0 == e1).astype(_f32)
        tr = lax.dot_general(idf, eye, (((0,), (0,)), ((), ())), precision=lax.Precision.HIGHEST)
        out[...] = tr.astype(_i32)


def _t2(sT):
    return pl.pallas_call(
        _t2_body,
        grid=(KP,),
        in_specs=[pl.BlockSpec((N2, B), lambda k: (0, 0))],
        out_specs=pl.BlockSpec((B, KP), lambda k: (0, 0)),
        out_shape=jax.ShapeDtypeStruct((B, KP), _i32),
        scratch_shapes=[pltpu.VMEM((N2, B), _f32),
                        pltpu.VMEM((KP, B), _i32)],
    )(sT)


def _g3_body(topk_hbm, clenr_hbm, c0_hbm, c1_hbm, erc_hbm, erc2_hbm,
             ea_hbm, eb_hbm, lensr_hbm,
             sidv, cb0, cb1, clrb, ebuf):
    b = _wid()
    pltpu.sync_copy(topk_hbm.at[b], sidv)
    pltpu.sync_copy(c0_hbm.at[sidv], cb0)
    pltpu.sync_copy(c1_hbm.at[sidv], cb1)
    pltpu.sync_copy(clenr_hbm.at[sidv], clrb)
    pltpu.sync_copy(clrb, lensr_hbm.at[b])

    @pl.loop(0, K)
    def _j(j):
        pltpu.sync_copy(erc_hbm.at[cb0.at[j]], ebuf)
        pltpu.sync_copy(ebuf, ea_hbm.at[b, pl.ds(j * LC, LC)])
        pltpu.sync_copy(erc2_hbm.at[cb1.at[j]], ebuf)
        pltpu.sync_copy(ebuf, eb_hbm.at[b, pl.ds(j * LC, LC)])


def _g3(topk, clen_rep, c0, c1, emb_rc, emb_rc2):
    out_type = (jax.ShapeDtypeStruct((B, KLC, D), _f32),
                jax.ShapeDtypeStruct((B, KLC, D), _f32),
                jax.ShapeDtypeStruct((B, KP, 16), _i32))
    return pl.kernel(
        _g3_body, out_type=out_type, mesh=_sc_mesh(),
        compiler_params=pltpu.CompilerParams(use_tc_tiling_on_sc=False),
        scratch_types=[pltpu.VMEM((KP,), _i32),
                       pltpu.VMEM((KP, LC), _i32),
                       pltpu.VMEM((KP, LC), _i32),
                       pltpu.VMEM((KP, 16), _i32),
                       pltpu.VMEM((LC, D), _f32)],
    )(topk, clen_rep, c0, c1, emb_rc, emb_rc2)


def _t3_body(ea, eb, w, vs, lens, a, out):
    h = ea[0] + eb[0]
    hw = jax.nn.relu(jnp.dot(h, w[...], preferred_element_type=_f32, precision=lax.Precision.HIGHEST))
    lg = jnp.sum(hw * vs[0], axis=1, keepdims=True)

    c1 = lax.broadcasted_iota(_i32, (1, KP), 1)
    lens50 = jnp.where(c1 < K, lens[0].astype(_f32), 0.0)
    u0 = lax.broadcasted_iota(_i32, (KP, KP), 0)
    u1 = lax.broadcasted_iota(_i32, (KP, KP), 1)
    ut = (u0 < u1).astype(_f32)
    off = lax.dot_general(lens50, ut, (((1,), (0,)), ((), ())), precision=lax.Precision.HIGHEST)

    r0 = lax.broadcasted_iota(_i32, (KLC, KP), 0) // LC
    rc = lax.broadcasted_iota(_i32, (KLC, KP), 1)
    P2 = (r0 == rc).astype(_f32)
    lens_row = lax.dot_general(P2, lens50, (((1,), (1,)), ((), ())), precision=lax.Precision.HIGHEST)
    off_row = lax.dot_general(P2, off, (((1,), (1,)), ((), ())), precision=lax.Precision.HIGHEST)
    tmod = (lax.broadcasted_iota(_i32, (KLC, 1), 0) % LC).astype(_f32)
    valid = tmod < lens_row
    pos = off_row + tmod

    ts = a[0, 0, 0] % KLC
    tsf = ts.astype(_f32)
    ctx = jnp.sum(lens50)
    hit = jnp.logical_and(valid, pos == tsf)
    pick = jnp.where(tsf < ctx, jnp.sum(jnp.where(hit, lg, 0.0)), -1e9)
    lm = jnp.max(jnp.where(valid, lg, -1e30))
    lse = lm + jnp.log(jnp.sum(jnp.where(valid, jnp.exp(lg - lm), 0.0)))

    bidx = pl.program_id(0)

    @pl.when(bidx == 0)
    def _():
        out[0, 0] = 0.0

    out[0, 0] += (lse - pick) * (1.0 / B)


def _t3(ea, eb, W, vs, lens, a):
    return pl.pallas_call(
        _t3_body,
        grid=(B,),
        in_specs=[pl.BlockSpec((1, KLC, D), lambda b: (b, 0, 0)),
                  pl.BlockSpec((1, KLC, D), lambda b: (b, 0, 0)),
                  pl.BlockSpec((D, D), lambda b: (0, 0)),
                  pl.BlockSpec((1, 1, D), lambda b: (b, 0, 0)),
                  pl.BlockSpec((1, 1, KP), lambda b: (b, 0, 0)),
                  pl.BlockSpec((1, 1, 16), lambda b: (b, 0, 0))],
        out_specs=pl.BlockSpec(memory_space=pltpu.SMEM),
        out_shape=jax.ShapeDtypeStruct((1, 1), _f32),
    )(ea, eb, W, vs.reshape(B, 1, D), lens.reshape(B, 1, KP),
      a.reshape(B, 1, 16))


def kernel(q, c, a, qlen, clen, alen, emb_ir, emb_rc, emb_rc2,
           W, Wq, w_s, w_e):
    q0f = q[:, :, 0].reshape(-1)
    q1f = q[:, :, 1].reshape(-1)
    c0 = c[:, :, 0]
    c1 = c[:, :, 1]
    qeir, qerca, qercb = _g1(q0f, q1f, emb_ir, emb_rc, emb_rc2)
    qv, vs = _t0(qeir, qerca, qercb, qlen, Wq, w_s)
    sT = _t1(emb_ir, qv)
    c0p = jnp.pad(c0, ((0, N2 - N), (0, 0)))
    clenp = jnp.pad(clen, (0, N2 - N), constant_values=1)
    scoresT = _g2(c0p.reshape(-1), clenp, sT)
    topk = _t2(scoresT)
    clen_rep = jnp.broadcast_to(clen[:, None], (N, 16))
    ea, eb, lensr = _g3(topk, clen_rep, c0, c1, emb_rc, emb_rc2)
    loss = _t3(ea, eb, W, vs, lensr[:, :, 0], a)
    return loss.reshape(())


# speedup vs baseline: 4.6214x; 1.2435x over previous
import functools

import jax
import jax.numpy as jnp
from jax import lax
from jax.experimental import pallas as pl
from jax.experimental.pallas import tpu as pltpu
from jax.experimental.pallas import tpu_sc as plsc

B = 32
LQ = 32
N = 10000
LC = 32
K = 50
KP = 64
D = 128
V = 30000
VP = 30080
N2 = 10240
KLC = K * LC

NC = 2
NS = 16
NW = NC * NS

_f32 = jnp.float32
_i32 = jnp.int32


def _wid():
    return lax.axis_index("s") * NC + lax.axis_index("c")


def _sc_mesh():
    return plsc.VectorSubcoreMesh(
        core_axis_name="c", subcore_axis_name="s",
        num_cores=NC, num_subcores=NS)


def _g1_body(q0f_hbm, q1f_hbm, eir_hbm, erc_hbm, erc2_hbm,
             qeir_hbm, qerca_hbm, qercb_hbm,
             t0v, t1v, ebuf):
    b = _wid()
    pltpu.sync_copy(q0f_hbm.at[pl.ds(b * LQ, LQ)], t0v)
    pltpu.sync_copy(q1f_hbm.at[pl.ds(b * LQ, LQ)], t1v)
    pltpu.sync_copy(eir_hbm.at[t0v], ebuf)
    pltpu.sync_copy(ebuf, qeir_hbm.at[b])
    pltpu.sync_copy(erc_hbm.at[t0v], ebuf)
    pltpu.sync_copy(ebuf, qerca_hbm.at[b])
    pltpu.sync_copy(erc2_hbm.at[t1v], ebuf)
    pltpu.sync_copy(ebuf, qercb_hbm.at[b])


def _g1(q0f, q1f, emb_ir, emb_rc, emb_rc2):
    out_type = (jax.ShapeDtypeStruct((B, LQ, D), _f32),
                jax.ShapeDtypeStruct((B, LQ, D), _f32),
                jax.ShapeDtypeStruct((B, LQ, D), _f32))
    return pl.kernel(
        _g1_body, out_type=out_type, mesh=_sc_mesh(),
        compiler_params=pltpu.CompilerParams(use_tc_tiling_on_sc=False),
        scratch_types=[pltpu.VMEM((LQ,), _i32),
                       pltpu.VMEM((LQ,), _i32),
                       pltpu.VMEM((LQ, D), _f32)],
    )(q0f, q1f, emb_ir, emb_rc, emb_rc2)


def _t0_body(qeir, qerca, qercb, qlen, wq, ws, qv_out, vs_out):
    R = B * LQ
    bi_b = lax.broadcasted_iota(_i32, (B, R), 0)
    bi_r = lax.broadcasted_iota(_i32, (B, R), 1) // LQ
    P = (bi_b == bi_r).astype(_f32)
    qlf = qlen[...].astype(_f32)
    ql_row = lax.dot_general(P, qlf, (((0,), (1,)), ((), ())), precision=lax.Precision.HIGHEST)
    tmod = (lax.broadcasted_iota(_i32, (R, 1), 0) % LQ).astype(_f32)
    m = (tmod < ql_row).astype(_f32)
    ql_col = lax.dot_general(P, m, (((1,), (0,)), ((), ())), precision=lax.Precision.HIGHEST)
    den = jnp.maximum(ql_col, 1.0)
    qv = lax.dot_general(P, qeir[...] * m, (((1,), (0,)), ((), ())), precision=lax.Precision.HIGHEST) / den
    qv_out[...] = qv
    erc = qerca[...] + qercb[...]
    qm2 = lax.dot_general(P, erc * m, (((1,), (0,)), ((), ())), precision=lax.Precision.HIGHEST) / den
    qh = jnp.dot(qm2, wq[...], preferred_element_type=_f32, precision=lax.Precision.HIGHEST)
    vs_out[...] = qh * ws[...]


def _t0(qeir, qerca, qercb, qlen, Wq, w_s):
    return pl.pallas_call(
        _t0_body,
        out_shape=(jax.ShapeDtypeStruct((B, D), _f32),
                   jax.ShapeDtypeStruct((B, D), _f32)),
    )(qeir.reshape(B * LQ, D), qerca.reshape(B * LQ, D),
      qercb.reshape(B * LQ, D), qlen.reshape(1, B),
      Wq, w_s.reshape(1, D))


_VB = 640


def _t1_body(emb, qv, out):
    i = pl.program_id(0)
    s = lax.dot_general(emb[...], qv[...], (((1,), (1,)), ((), ())), precision=lax.Precision.HIGHEST)
    row = lax.broadcasted_iota(_i32, (_VB, 1), 0) + i * _VB
    out[...] = jnp.where(row < V, s, 0.0)


def _t1(emb_ir, qv):
    return pl.pallas_call(
        _t1_body,
        grid=(VP // _VB,),
        in_specs=[pl.BlockSpec((_VB, D), lambda i: (i, 0)),
                  pl.BlockSpec((B, D), lambda i: (0, 0))],
        out_specs=pl.BlockSpec((_VB, B), lambda i: (i, 0)),
        out_shape=jax.ShapeDtypeStruct((VP, B), _f32),
    )(emb_ir, qv)


_NSW = N2 // NW
_NSUB = _NSW // 4


def _g2_body(c0f_hbm, clen_hbm, sT_hbm, out_hbm,
             cbuf, clv, tokb, srows, sbuf, sT_sh, sem0, sem1):
    w = _wid()
    sems = (sem0, sem1)

    @pl.when(lax.axis_index("s") == 0)
    def _():
        pltpu.sync_copy(sT_hbm, sT_sh)
    plsc.subcore_barrier()

    pltpu.sync_copy(c0f_hbm.at[pl.ds(w * _NSW * LC, _NSW * LC)], cbuf)
    pltpu.sync_copy(clen_hbm.at[pl.ds(w * _NSW, _NSW)], clv.at[pl.ds(0, _NSW)])

    def build_and_issue(s, par):
        clvvec = clv[pl.ds(s * 4, 16)]
        for r in range(4):
            cls = jnp.broadcast_to(clvvec[r], (16,))
            for k in range(2):
                tl = lax.iota(_i32, 16) + 16 * k
                tok = cbuf[pl.ds((s * 4 + r) * LC + 16 * k, 16)]
                tokb[par, pl.ds(r * LC + 16 * k, 16)] = (
                    jnp.where(tl < cls, tok, V))
        pltpu.async_copy(sT_sh.at[tokb.at[par]], srows.at[par], sems[par])

    def wait(par):
        pltpu.make_async_copy(
            sT_sh.at[tokb.at[par]], srows.at[par], sems[par]).wait()

    def accumulate(s, par):
        clvvec = clv[pl.ds(s * 4, 16)]
        for r in range(4):
            r0 = r * LC
            a0 = srows[par, r0, pl.ds(0, 16)]
            a1 = srows[par, r0, pl.ds(16, 16)]
            for t in range(1, LC):
                a0 = a0 + srows[par, r0 + t, pl.ds(0, 16)]
                a1 = a1 + srows[par, r0 + t, pl.ds(16, 16)]
            clf = jnp.broadcast_to(clvvec[r], (16,)).astype(_f32)
            sbuf[s * 4 + r, pl.ds(0, 16)] = a0 / clf
            sbuf[s * 4 + r, pl.ds(16, 16)] = a1 / clf

    build_and_issue(0, 0)

    @pl.loop(0, _NSUB // 2)
    def _i(i):
        for par in range(2):
            s = i * 2 + par

            @pl.when(s < _NSUB - 1)
            def _():
                build_and_issue(s + 1, 1 - par)

            wait(par)
            accumulate(s, par)

    pltpu.sync_copy(sbuf, out_hbm.at[pl.ds(w * _NSW, _NSW)])


def _g2(c0f, clen, sT):
    return pl.kernel(
        _g2_body, out_type=jax.ShapeDtypeStruct((N2, B), _f32),
        mesh=_sc_mesh(),
        compiler_params=pltpu.CompilerParams(use_tc_tiling_on_sc=False),
        scratch_types=[pltpu.VMEM((_NSW * LC,), _i32),
                       pltpu.VMEM((_NSW + 16,), _i32),
                       pltpu.VMEM((2, 128), _i32),
                       pltpu.VMEM((2, 128, B), _f32),
                       pltpu.VMEM((_NSW, B), _f32),
                       pltpu.VMEM_SHARED((VP, B), _f32),
                       pltpu.SemaphoreType.DMA,
                       pltpu.SemaphoreType.DMA],
    )(c0f, clen, sT)


def _t2_body(sT, out, s_scr, idx_scr):
    k = pl.program_id(0)
    bi0 = lax.broadcasted_iota(_i32, (N2, B), 0)

    @pl.when(k == 0)
    def _():
        s_scr[...] = jnp.where(bi0 < N, sT[...], -1e30)
        idx_scr[...] = jnp.zeros((KP, B), _i32)

    s = s_scr[...]
    m = jnp.max(s, axis=0, keepdims=True)
    idx = jnp.min(jnp.where(s == m, bi0, N2), axis=0, keepdims=True)
    idx_scr[pl.ds(k, 1), :] = idx
    s_scr[...] = jnp.where(bi0 == idx, -jnp.inf, s)

    @pl.when(k == K - 1)
    def _():
        idf = idx_scr[...].astype(_f32)
        e0 = lax.broadcasted_iota(_i32, (KP, KP), 0)
        e1 = lax.broadcasted_iota(_i32, (KP, KP), 1)
        eye = (e0 == e1).astype(_f32)
        tr = lax.dot_general(idf, eye, (((0,), (0,)), ((), ())), precision=lax.Precision.HIGHEST)
        out[...] = tr.astype(_i32)


def _t2(sT):
    return pl.pallas_call(
        _t2_body,
        grid=(K,),
        in_specs=[pl.BlockSpec((N2, B), lambda k: (0, 0))],
        out_specs=pl.BlockSpec((B, KP), lambda k: (0, 0)),
        out_shape=jax.ShapeDtypeStruct((B, KP), _i32),
        scratch_shapes=[pltpu.VMEM((N2, B), _f32),
                        pltpu.VMEM((KP, B), _i32)],
    )(sT)


def _g3_body(topk_hbm, clenr_hbm, c0_hbm, c1_hbm, erc_hbm, erc2_hbm,
             ea_hbm, eb_hbm, lensr_hbm,
             sidv, cb0, cb1, clrb, ebA, ebB,
             siA0, siA1, siB0, siB1, soA0, soA1, soB0, soB1):
    b = _wid()
    semInA = (siA0, siA1)
    semInB = (siB0, siB1)
    semOutA = (soA0, soA1)
    semOutB = (soB0, soB1)
    pltpu.sync_copy(topk_hbm.at[b], sidv)
    pltpu.sync_copy(c0_hbm.at[sidv], cb0)
    pltpu.sync_copy(c1_hbm.at[sidv], cb1)
    pltpu.sync_copy(clenr_hbm.at[sidv], clrb)
    pltpu.sync_copy(clrb, lensr_hbm.at[b])

    def issue_in(j, par):
        pltpu.async_copy(erc_hbm.at[cb0.at[j]], ebA.at[par], semInA[par])
        pltpu.async_copy(erc2_hbm.at[cb1.at[j]], ebB.at[par], semInB[par])

    def wait_in(j, par):
        pltpu.make_async_copy(
            erc_hbm.at[cb0.at[j]], ebA.at[par], semInA[par]).wait()
        pltpu.make_async_copy(
            erc2_hbm.at[cb1.at[j]], ebB.at[par], semInB[par]).wait()

    def issue_out(j, par):
        pltpu.async_copy(
            ebA.at[par], ea_hbm.at[b, pl.ds(j * LC, LC)], semOutA[par])
        pltpu.async_copy(
            ebB.at[par], eb_hbm.at[b, pl.ds(j * LC, LC)], semOutB[par])

    def wait_out(j, par):
        pltpu.make_async_copy(
            ebA.at[par], ea_hbm.at[b, pl.ds(j * LC, LC)], semOutA[par]).wait()
        pltpu.make_async_copy(
            ebB.at[par], eb_hbm.at[b, pl.ds(j * LC, LC)], semOutB[par]).wait()

    issue_in(0, 0)

    @pl.loop(0, K // 2)
    def _i(i):
        for par in range(2):
            j = i * 2 + par

            @pl.when(j < K - 1)
            def _():
                @pl.when(j >= 1)
                def _():
                    wait_out(j - 1, 1 - par)
                issue_in(j + 1, 1 - par)

            wait_in(j, par)
            issue_out(j, par)

    wait_out(K - 2, 0)
    wait_out(K - 1, 1)


def _g3(topk, clen_rep, c0, c1, emb_rc, emb_rc2):
    out_type = (jax.ShapeDtypeStruct((B, KLC, D), _f32),
                jax.ShapeDtypeStruct((B, KLC, D), _f32),
                jax.ShapeDtypeStruct((B, KP, 16), _i32))
    return pl.kernel(
        _g3_body, out_type=out_type, mesh=_sc_mesh(),
        compiler_params=pltpu.CompilerParams(use_tc_tiling_on_sc=False),
        scratch_types=[pltpu.VMEM((KP,), _i32),
                       pltpu.VMEM((KP, LC), _i32),
                       pltpu.VMEM((KP, LC), _i32),
                       pltpu.VMEM((KP, 16), _i32),
                       pltpu.VMEM((2, LC, D), _f32),
                       pltpu.VMEM((2, LC, D), _f32)]
                      + [pltpu.SemaphoreType.DMA] * 8,
    )(topk, clen_rep, c0, c1, emb_rc, emb_rc2)


def _t3_body(ea, eb, w, vs, lens, a, out):
    h = ea[0] + eb[0]
    hw = jax.nn.relu(jnp.dot(h, w[...], preferred_element_type=_f32, precision=lax.Precision.HIGHEST))
    lg = jnp.sum(hw * vs[0], axis=1, keepdims=True)

    c1 = lax.broadcasted_iota(_i32, (1, KP), 1)
    lens50 = jnp.where(c1 < K, lens[0].astype(_f32), 0.0)
    u0 = lax.broadcasted_iota(_i32, (KP, KP), 0)
    u1 = lax.broadcasted_iota(_i32, (KP, KP), 1)
    ut = (u0 < u1).astype(_f32)
    off = lax.dot_general(lens50, ut, (((1,), (0,)), ((), ())), precision=lax.Precision.HIGHEST)

    r0 = lax.broadcasted_iota(_i32, (KLC, KP), 0) // LC
    rc = lax.broadcasted_iota(_i32, (KLC, KP), 1)
    P2 = (r0 == rc).astype(_f32)
    lens_row = lax.dot_general(P2, lens50, (((1,), (1,)), ((), ())), precision=lax.Precision.HIGHEST)
    off_row = lax.dot_general(P2, off, (((1,), (1,)), ((), ())), precision=lax.Precision.HIGHEST)
    tmod = (lax.broadcasted_iota(_i32, (KLC, 1), 0) % LC).astype(_f32)
    valid = tmod < lens_row
    pos = off_row + tmod

    ts = a[0, 0, 0] % KLC
    tsf = ts.astype(_f32)
    ctx = jnp.sum(lens50)
    hit = jnp.logical_and(valid, pos == tsf)
    pick = jnp.where(tsf < ctx, jnp.sum(jnp.where(hit, lg, 0.0)), -1e9)
    lm = jnp.max(jnp.where(valid, lg, -1e30))
    lse = lm + jnp.log(jnp.sum(jnp.where(valid, jnp.exp(lg - lm), 0.0)))

    bidx = pl.program_id(0)

    @pl.when(bidx == 0)
    def _():
        out[0, 0] = 0.0

    out[0, 0] += (lse - pick) * (1.0 / B)


def _t3(ea, eb, W, vs, lens, a):
    return pl.pallas_call(
        _t3_body,
        grid=(B,),
        in_specs=[pl.BlockSpec((1, KLC, D), lambda b: (b, 0, 0)),
                  pl.BlockSpec((1, KLC, D), lambda b: (b, 0, 0)),
                  pl.BlockSpec((D, D), lambda b: (0, 0)),
                  pl.BlockSpec((1, 1, D), lambda b: (b, 0, 0)),
                  pl.BlockSpec((1, 1, KP), lambda b: (b, 0, 0)),
                  pl.BlockSpec((1, 1, 16), lambda b: (b, 0, 0))],
        out_specs=pl.BlockSpec(memory_space=pltpu.SMEM),
        out_shape=jax.ShapeDtypeStruct((1, 1), _f32),
    )(ea, eb, W, vs.reshape(B, 1, D), lens.reshape(B, 1, KP),
      a.reshape(B, 1, 16))


def kernel(q, c, a, qlen, clen, alen, emb_ir, emb_rc, emb_rc2,
           W, Wq, w_s, w_e):
    q0f = q[:, :, 0].reshape(-1)
    q1f = q[:, :, 1].reshape(-1)
    c0 = c[:, :, 0]
    c1 = c[:, :, 1]
    qeir, qerca, qercb = _g1(q0f, q1f, emb_ir, emb_rc, emb_rc2)
    qv, vs = _t0(qeir, qerca, qercb, qlen, Wq, w_s)
    sT = _t1(emb_ir, qv)
    c0p = jnp.pad(c0, ((0, N2 - N), (0, 0)))
    clenp = jnp.pad(clen, (0, N2 - N), constant_values=1)
    scoresT = _g2(c0p.reshape(-1), clenp, sT)
    topk = _t2(scoresT)
    clen_rep = jnp.broadcast_to(clen[:, None], (N, 16))
    ea, eb, lensr = _g3(topk, clen_rep, c0, c1, emb_rc, emb_rc2)
    loss = _t3(ea, eb, W, vs, lensr[:, :, 0], a)
    return loss.reshape(())


# speedup vs baseline: 5.9790x; 1.2938x over previous
import functools

import jax
import jax.numpy as jnp
from jax import lax
from jax.experimental import pallas as pl
from jax.experimental.pallas import tpu as pltpu
from jax.experimental.pallas import tpu_sc as plsc

B = 32
LQ = 32
N = 10000
LC = 32
K = 50
KP = 64
D = 128
V = 30000
VP = 30080
N2 = 10240
KLC = K * LC

NC = 2
NS = 16
NW = NC * NS

_f32 = jnp.float32
_i32 = jnp.int32


def _wid():
    return lax.axis_index("s") * NC + lax.axis_index("c")


def _sc_mesh():
    return plsc.VectorSubcoreMesh(
        core_axis_name="c", subcore_axis_name="s",
        num_cores=NC, num_subcores=NS)


def _g1_body(q0f_hbm, q1f_hbm, eir_hbm, erc_hbm, erc2_hbm,
             qeir_hbm, qerca_hbm, qercb_hbm,
             t0v, t1v, ebuf):
    b = _wid()
    pltpu.sync_copy(q0f_hbm.at[pl.ds(b * LQ, LQ)], t0v)
    pltpu.sync_copy(q1f_hbm.at[pl.ds(b * LQ, LQ)], t1v)
    pltpu.sync_copy(eir_hbm.at[t0v], ebuf)
    pltpu.sync_copy(ebuf, qeir_hbm.at[b])
    pltpu.sync_copy(erc_hbm.at[t0v], ebuf)
    pltpu.sync_copy(ebuf, qerca_hbm.at[b])
    pltpu.sync_copy(erc2_hbm.at[t1v], ebuf)
    pltpu.sync_copy(ebuf, qercb_hbm.at[b])


def _g1(q0f, q1f, emb_ir, emb_rc, emb_rc2):
    out_type = (jax.ShapeDtypeStruct((B, LQ, D), _f32),
                jax.ShapeDtypeStruct((B, LQ, D), _f32),
                jax.ShapeDtypeStruct((B, LQ, D), _f32))
    return pl.kernel(
        _g1_body, out_type=out_type, mesh=_sc_mesh(),
        compiler_params=pltpu.CompilerParams(use_tc_tiling_on_sc=False),
        scratch_types=[pltpu.VMEM((LQ,), _i32),
                       pltpu.VMEM((LQ,), _i32),
                       pltpu.VMEM((LQ, D), _f32)],
    )(q0f, q1f, emb_ir, emb_rc, emb_rc2)


def _t0_body(qeir, qerca, qercb, qlen, wq, ws, qv_out, vs_out):
    R = B * LQ
    bi_b = lax.broadcasted_iota(_i32, (B, R), 0)
    bi_r = lax.broadcasted_iota(_i32, (B, R), 1) // LQ
    P = (bi_b == bi_r).astype(_f32)
    qlf = qlen[...].astype(_f32)
    ql_row = lax.dot_general(P, qlf, (((0,), (1,)), ((), ())), precision=lax.Precision.HIGHEST)
    tmod = (lax.broadcasted_iota(_i32, (R, 1), 0) % LQ).astype(_f32)
    m = (tmod < ql_row).astype(_f32)
    ql_col = lax.dot_general(P, m, (((1,), (0,)), ((), ())), precision=lax.Precision.HIGHEST)
    den = jnp.maximum(ql_col, 1.0)
    qv = lax.dot_general(P, qeir[...] * m, (((1,), (0,)), ((), ())), precision=lax.Precision.HIGHEST) / den
    qv_out[...] = qv
    erc = qerca[...] + qercb[...]
    qm2 = lax.dot_general(P, erc * m, (((1,), (0,)), ((), ())), precision=lax.Precision.HIGHEST) / den
    qh = jnp.dot(qm2, wq[...], preferred_element_type=_f32, precision=lax.Precision.HIGHEST)
    vs_out[...] = qh * ws[...]


def _t0(qeir, qerca, qercb, qlen, Wq, w_s):
    return pl.pallas_call(
        _t0_body,
        out_shape=(jax.ShapeDtypeStruct((B, D), _f32),
                   jax.ShapeDtypeStruct((B, D), _f32)),
    )(qeir.reshape(B * LQ, D), qerca.reshape(B * LQ, D),
      qercb.reshape(B * LQ, D), qlen.reshape(1, B),
      Wq, w_s.reshape(1, D))


_VB = 640


def _t1_body(emb, qv, out):
    i = pl.program_id(0)
    s = lax.dot_general(emb[...], qv[...], (((1,), (1,)), ((), ())), precision=lax.Precision.HIGHEST)
    row = lax.broadcasted_iota(_i32, (_VB, 1), 0) + i * _VB
    out[...] = jnp.where(row < V, s, 0.0)


def _t1(emb_ir, qv):
    return pl.pallas_call(
        _t1_body,
        grid=(VP // _VB,),
        in_specs=[pl.BlockSpec((_VB, D), lambda i: (i, 0)),
                  pl.BlockSpec((B, D), lambda i: (0, 0))],
        out_specs=pl.BlockSpec((_VB, B), lambda i: (i, 0)),
        out_shape=jax.ShapeDtypeStruct((VP, B), _f32),
    )(emb_ir, qv)


_NSW = N2 // NW
_NSUB = _NSW // 4


def _g2_body(c0f_hbm, clen_hbm, sT_hbm, out_hbm,
             cbuf, clv, tokb, srows, sbuf, sT_sh, sem0, sem1):
    w = _wid()
    sems = (sem0, sem1)

    @pl.when(lax.axis_index("s") == 0)
    def _():
        pltpu.sync_copy(sT_hbm, sT_sh)
    plsc.subcore_barrier()

    pltpu.sync_copy(c0f_hbm.at[pl.ds(w * _NSW * LC, _NSW * LC)], cbuf)
    pltpu.sync_copy(clen_hbm.at[pl.ds(w * _NSW, _NSW)], clv.at[pl.ds(0, _NSW)])

    def build_and_issue(s, par):
        clvvec = clv[pl.ds(s * 4, 16)]
        for r in range(4):
            cls = jnp.broadcast_to(clvvec[r], (16,))
            for k in range(2):
                tl = lax.iota(_i32, 16) + 16 * k
                tok = cbuf[pl.ds((s * 4 + r) * LC + 16 * k, 16)]
                tokb[par, pl.ds(r * LC + 16 * k, 16)] = (
                    jnp.where(tl < cls, tok, V))
        pltpu.async_copy(sT_sh.at[tokb.at[par]], srows.at[par], sems[par])

    def wait(par):
        pltpu.make_async_copy(
            sT_sh.at[tokb.at[par]], srows.at[par], sems[par]).wait()

    def accumulate(s, par):
        clvvec = clv[pl.ds(s * 4, 16)]
        for r in range(4):
            r0 = r * LC
            a0 = srows[par, r0, pl.ds(0, 16)]
            a1 = srows[par, r0, pl.ds(16, 16)]
            for t in range(1, LC):
                a0 = a0 + srows[par, r0 + t, pl.ds(0, 16)]
                a1 = a1 + srows[par, r0 + t, pl.ds(16, 16)]
            clf = jnp.broadcast_to(clvvec[r], (16,)).astype(_f32)
            sbuf[s * 4 + r, pl.ds(0, 16)] = a0 / clf
            sbuf[s * 4 + r, pl.ds(16, 16)] = a1 / clf

    build_and_issue(0, 0)

    @pl.loop(0, _NSUB // 2)
    def _i(i):
        for par in range(2):
            s = i * 2 + par

            @pl.when(s < _NSUB - 1)
            def _():
                build_and_issue(s + 1, 1 - par)

            wait(par)
            accumulate(s, par)

    pltpu.sync_copy(sbuf, out_hbm.at[pl.ds(w * _NSW, _NSW)])


def _g2(c0f, clen, sT):
    return pl.kernel(
        _g2_body, out_type=jax.ShapeDtypeStruct((N2, B), _f32),
        mesh=_sc_mesh(),
        compiler_params=pltpu.CompilerParams(use_tc_tiling_on_sc=False),
        scratch_types=[pltpu.VMEM((_NSW * LC,), _i32),
                       pltpu.VMEM((_NSW + 16,), _i32),
                       pltpu.VMEM((2, 128), _i32),
                       pltpu.VMEM((2, 128, B), _f32),
                       pltpu.VMEM((_NSW, B), _f32),
                       pltpu.VMEM_SHARED((VP, B), _f32),
                       pltpu.SemaphoreType.DMA,
                       pltpu.SemaphoreType.DMA],
    )(c0f, clen, sT)


def _t2_body(sT, out, s2, idx_scr):
    k = pl.program_id(0)
    bi_n = lax.broadcasted_iota(_i32, (B, N2), 1)

    @pl.when(k == 0)
    def _():
        s2[...] = jnp.where(bi_n < N, jnp.transpose(sT[...]), -1e30)
        idx_scr[...] = jnp.zeros((B, KP), _i32)

    s = s2[...]
    m = jnp.max(s, axis=1, keepdims=True)
    idx = jnp.min(jnp.where(s == m, bi_n, N2), axis=1, keepdims=True)
    bi_k = lax.broadcasted_iota(_i32, (B, KP), 1)
    idx_scr[...] = jnp.where(bi_k == k, idx, idx_scr[...])
    s2[...] = jnp.where(bi_n == idx, -jnp.inf, s)

    @pl.when(k == K - 1)
    def _():
        out[...] = idx_scr[...]


def _t2(sT):
    return pl.pallas_call(
        _t2_body,
        grid=(K,),
        in_specs=[pl.BlockSpec((N2, B), lambda k: (0, 0))],
        out_specs=pl.BlockSpec((B, KP), lambda k: (0, 0)),
        out_shape=jax.ShapeDtypeStruct((B, KP), _i32),
        scratch_shapes=[pltpu.VMEM((B, N2), _f32),
                        pltpu.VMEM((B, KP), _i32)],
    )(sT)


def _g3_body(topk_hbm, clenr_hbm, c0_hbm, c1_hbm, erc_hbm, erc2_hbm,
             ea_hbm, eb_hbm, lensr_hbm,
             sidv, cb0, cb1, clrb, ebA, ebB,
             siA0, siA1, siB0, siB1, soA0, soA1, soB0, soB1):
    b = _wid()
    semInA = (siA0, siA1)
    semInB = (siB0, siB1)
    semOutA = (soA0, soA1)
    semOutB = (soB0, soB1)
    pltpu.sync_copy(topk_hbm.at[b], sidv)
    pltpu.sync_copy(c0_hbm.at[sidv], cb0)
    pltpu.sync_copy(c1_hbm.at[sidv], cb1)
    pltpu.sync_copy(clenr_hbm.at[sidv], clrb)
    pltpu.sync_copy(clrb, lensr_hbm.at[b])

    def issue_in(j, par):
        pltpu.async_copy(erc_hbm.at[cb0.at[j]], ebA.at[par], semInA[par])
        pltpu.async_copy(erc2_hbm.at[cb1.at[j]], ebB.at[par], semInB[par])

    def wait_in(j, par):
        pltpu.make_async_copy(
            erc_hbm.at[cb0.at[j]], ebA.at[par], semInA[par]).wait()
        pltpu.make_async_copy(
            erc2_hbm.at[cb1.at[j]], ebB.at[par], semInB[par]).wait()

    def issue_out(j, par):
        pltpu.async_copy(
            ebA.at[par], ea_hbm.at[b, pl.ds(j * LC, LC)], semOutA[par])
        pltpu.async_copy(
            ebB.at[par], eb_hbm.at[b, pl.ds(j * LC, LC)], semOutB[par])

    def wait_out(j, par):
        pltpu.make_async_copy(
            ebA.at[par], ea_hbm.at[b, pl.ds(j * LC, LC)], semOutA[par]).wait()
        pltpu.make_async_copy(
            ebB.at[par], eb_hbm.at[b, pl.ds(j * LC, LC)], semOutB[par]).wait()

    issue_in(0, 0)

    @pl.loop(0, K // 2)
    def _i(i):
        for par in range(2):
            j = i * 2 + par

            @pl.when(j < K - 1)
            def _():
                @pl.when(j >= 1)
                def _():
                    wait_out(j - 1, 1 - par)
                issue_in(j + 1, 1 - par)

            wait_in(j, par)
            issue_out(j, par)

    wait_out(K - 2, 0)
    wait_out(K - 1, 1)


def _g3(topk, clen_rep, c0, c1, emb_rc, emb_rc2):
    out_type = (jax.ShapeDtypeStruct((B, KLC, D), _f32),
                jax.ShapeDtypeStruct((B, KLC, D), _f32),
                jax.ShapeDtypeStruct((B, KP, 16), _i32))
    return pl.kernel(
        _g3_body, out_type=out_type, mesh=_sc_mesh(),
        compiler_params=pltpu.CompilerParams(use_tc_tiling_on_sc=False),
        scratch_types=[pltpu.VMEM((KP,), _i32),
                       pltpu.VMEM((KP, LC), _i32),
                       pltpu.VMEM((KP, LC), _i32),
                       pltpu.VMEM((KP, 16), _i32),
                       pltpu.VMEM((2, LC, D), _f32),
                       pltpu.VMEM((2, LC, D), _f32)]
                      + [pltpu.SemaphoreType.DMA] * 8,
    )(topk, clen_rep, c0, c1, emb_rc, emb_rc2)


def _t3_body(ea, eb, w, vs, lens, a, out):
    h = ea[0] + eb[0]
    hw = jax.nn.relu(jnp.dot(h, w[...], preferred_element_type=_f32, precision=lax.Precision.HIGHEST))
    lg = jnp.sum(hw * vs[0], axis=1, keepdims=True)

    c1 = lax.broadcasted_iota(_i32, (1, KP), 1)
    lens50 = jnp.where(c1 < K, lens[0].astype(_f32), 0.0)
    u0 = lax.broadcasted_iota(_i32, (KP, KP), 0)
    u1 = lax.broadcasted_iota(_i32, (KP, KP), 1)
    ut = (u0 < u1).astype(_f32)
    off = lax.dot_general(lens50, ut, (((1,), (0,)), ((), ())), precision=lax.Precision.HIGHEST)

    r0 = lax.broadcasted_iota(_i32, (KLC, KP), 0) // LC
    rc = lax.broadcasted_iota(_i32, (KLC, KP), 1)
    P2 = (r0 == rc).astype(_f32)
    lens_row = lax.dot_general(P2, lens50, (((1,), (1,)), ((), ())), precision=lax.Precision.HIGHEST)
    off_row = lax.dot_general(P2, off, (((1,), (1,)), ((), ())), precision=lax.Precision.HIGHEST)
    tmod = (lax.broadcasted_iota(_i32, (KLC, 1), 0) % LC).astype(_f32)
    valid = tmod < lens_row
    pos = off_row + tmod

    ts = a[0, 0, 0] % KLC
    tsf = ts.astype(_f32)
    ctx = jnp.sum(lens50)
    hit = jnp.logical_and(valid, pos == tsf)
    pick = jnp.where(tsf < ctx, jnp.sum(jnp.where(hit, lg, 0.0)), -1e9)
    lm = jnp.max(jnp.where(valid, lg, -1e30))
    lse = lm + jnp.log(jnp.sum(jnp.where(valid, jnp.exp(lg - lm), 0.0)))

    bidx = pl.program_id(0)

    @pl.when(bidx == 0)
    def _():
        out[0, 0] = 0.0

    out[0, 0] += (lse - pick) * (1.0 / B)


def _t3(ea, eb, W, vs, lens, a):
    return pl.pallas_call(
        _t3_body,
        grid=(B,),
        in_specs=[pl.BlockSpec((1, KLC, D), lambda b: (b, 0, 0)),
                  pl.BlockSpec((1, KLC, D), lambda b: (b, 0, 0)),
                  pl.BlockSpec((D, D), lambda b: (0, 0)),
                  pl.BlockSpec((1, 1, D), lambda b: (b, 0, 0)),
                  pl.BlockSpec((1, 1, KP), lambda b: (b, 0, 0)),
                  pl.BlockSpec((1, 1, 16), lambda b: (b, 0, 0))],
        out_specs=pl.BlockSpec(memory_space=pltpu.SMEM),
        out_shape=jax.ShapeDtypeStruct((1, 1), _f32),
    )(ea, eb, W, vs.reshape(B, 1, D), lens.reshape(B, 1, KP),
      a.reshape(B, 1, 16))


def kernel(q, c, a, qlen, clen, alen, emb_ir, emb_rc, emb_rc2,
           W, Wq, w_s, w_e):
    q0f = q[:, :, 0].reshape(-1)
    q1f = q[:, :, 1].reshape(-1)
    c0 = c[:, :, 0]
    c1 = c[:, :, 1]
    qeir, qerca, qercb = _g1(q0f, q1f, emb_ir, emb_rc, emb_rc2)
    qv, vs = _t0(qeir, qerca, qercb, qlen, Wq, w_s)
    sT = _t1(emb_ir, qv)
    c0p = jnp.pad(c0, ((0, N2 - N), (0, 0)))
    clenp = jnp.pad(clen, (0, N2 - N), constant_values=1)
    scoresT = _g2(c0p.reshape(-1), clenp, sT)
    topk = _t2(scoresT)
    clen_rep = jnp.broadcast_to(clen[:, None], (N, 16))
    ea, eb, lensr = _g3(topk, clen_rep, c0, c1, emb_rc, emb_rc2)
    loss = _t3(ea, eb, W, vs, lensr[:, :, 0], a)
    return loss.reshape(())
